# baseline (speedup 1.0000x reference)
import functools

import jax
import jax.numpy as jnp
from jax import lax
from jax.experimental import pallas as pl
from jax.experimental.pallas import tpu as pltpu
from jax.experimental.pallas import tpu_sc as plsc

N = 10000
E = 160000
D_IN = 256
D_HID = 256
D_OUT = 128
NC, NS = 2, 16
RPS = 632
NPAD = RPS * NS
BATCH = 128
NSUB = 4
NB2 = 40
BM = 1000


@functools.lru_cache(maxsize=None)
def _make_segsum(nb):
    mesh = plsc.VectorSubcoreMesh(core_axis_name="c", subcore_axis_name="s",
                                  num_cores=NC, num_subcores=NS)
    sub = BATCH // NSUB

    def body(table, srcp, dstp, zeros, out, src_v, dst_v, rows_a, rows_b,
             acc, sem_a, sem_b):
        c = lax.axis_index("c")
        s = lax.axis_index("s")
        sl = pl.ds(s * RPS, RPS)
        pltpu.sync_copy(srcp.at[c, s], src_v)
        pltpu.sync_copy(dstp.at[c, s], dst_v)
        pltpu.sync_copy(zeros.at[sl], acc.at[sl])
        plsc.subcore_barrier()

        def gather_start(j, buf, sem):
            for k in range(NSUB):
                ds = pl.ds(k * sub, sub)
                pltpu.async_copy(table.at[src_v.at[j, ds]], buf.at[ds], sem)

        def gather_wait(j, buf, sem):
            pltpu.make_async_copy(table.at[src_v.at[j]], buf, sem).wait()

        gather_start(0, rows_a, sem_a)

        def step(i, carry):
            ja = 2 * i
            jb = ja + 1
            gather_start(jb, rows_b, sem_b)
            gather_wait(ja, rows_a, sem_a)
            pltpu.sync_copy(rows_a, acc.at[dst_v.at[ja]], add=True)

            @pl.when(ja + 2 < nb)
            def _():
                gather_start(ja + 2, rows_a, sem_a)

            gather_wait(jb, rows_b, sem_b)
            pltpu.sync_copy(rows_b, acc.at[dst_v.at[jb]], add=True)
            return carry

        lax.fori_loop(0, nb // 2, step, 0)
        plsc.subcore_barrier()
        pltpu.sync_copy(acc.at[sl], out.at[c, sl])

    return pl.kernel(
        body,
        out_type=[jax.ShapeDtypeStruct((NC, NPAD, 128), jnp.float32)],
        mesh=mesh,
        scratch_types=[
            pltpu.VMEM((nb, BATCH), jnp.int32),
            pltpu.VMEM((nb, BATCH), jnp.int32),
            pltpu.VMEM((BATCH, 128), jnp.float32),
            pltpu.VMEM((BATCH, 128), jnp.float32),
            pltpu.VMEM_SHARED((NPAD, 128), jnp.float32),
            pltpu.SemaphoreType.DMA,
            pltpu.SemaphoreType.DMA,
        ])


@functools.lru_cache(maxsize=None)
def _make_deg():
    mesh = plsc.VectorSubcoreMesh(core_axis_name="c", subcore_axis_name="s",
                                  num_cores=NC, num_subcores=NS)

    def body(dstp, zeros, ones, degp, dst_v, ones_v, dacc):
        c = lax.axis_index("c")
        s = lax.axis_index("s")
        sl = pl.ds(s * RPS, RPS)
        pltpu.sync_copy(dstp.at[c, s], dst_v)
        pltpu.sync_copy(ones, ones_v)
        pltpu.sync_copy(zeros.at[sl], dacc.at[sl])
        plsc.subcore_barrier()

        def step(j, carry):
            pltpu.sync_copy(ones_v, dacc.at[dst_v.at[j]], add=True)
            return carry

        lax.fori_loop(0, NB2, step, 0)
        plsc.subcore_barrier()
        pltpu.sync_copy(dacc.at[sl], degp.at[c, sl])

    return pl.kernel(
        body,
        out_type=[jax.ShapeDtypeStruct((NC, NPAD, 128), jnp.float32)],
        mesh=mesh,
        scratch_types=[
            pltpu.VMEM((NB2, BATCH), jnp.int32),
            pltpu.VMEM((BATCH, 128), jnp.float32),
            pltpu.VMEM_SHARED((NPAD, 128), jnp.float32),
        ])


def _tc1(x, wcat, b1):
    def body(x_ref, w_ref, b_ref, u_ref, v_ref):
        y = jnp.dot(x_ref[...], w_ref[...],
                    preferred_element_type=jnp.float32)
        u_ref[0] = y[:, :128]
        u_ref[1] = y[:, 128:256]
        v_ref[...] = y[:, 256:] + b_ref[...]

    return pl.pallas_call(
        body,
        grid=(N // BM,),
        in_specs=[
            pl.BlockSpec((BM, D_IN), lambda i: (i, 0)),
            pl.BlockSpec((D_IN, 2 * D_HID), lambda i: (0, 0)),
            pl.BlockSpec((1, D_HID), lambda i: (0, 0)),
        ],
        out_specs=[
            pl.BlockSpec((2, BM, 128), lambda i: (0, i, 0)),
            pl.BlockSpec((BM, D_HID), lambda i: (i, 0)),
        ],
        out_shape=[
            jax.ShapeDtypeStruct((2, N, 128), jnp.float32),
            jax.ShapeDtypeStruct((N, D_HID), jnp.float32),
        ],
    )(x, wcat, b1)


def _tc2(s1a, s1b, degp, v1, wcat, b2):
    def body(s1a_ref, s1b_ref, deg_ref, v1_ref, w_ref, b_ref,
             u2_ref, v2_ref):
        deg = jnp.maximum(deg_ref[0, :, :1] + deg_ref[1, :, :1], 1.0)
        agg = jnp.concatenate([s1a_ref[0] + s1a_ref[1],
                               s1b_ref[0] + s1b_ref[1]], axis=1) / deg
        h = jnp.maximum(agg + v1_ref[...], 0.0)
        y = jnp.dot(h, w_ref[...], preferred_element_type=jnp.float32)
        u2_ref[...] = y[:, :D_OUT]
        v2_ref[...] = y[:, D_OUT:] + b_ref[...]

    return pl.pallas_call(
        body,
        grid=(N // BM,),
        in_specs=[
            pl.BlockSpec((2, BM, 128), lambda i: (0, i, 0)),
            pl.BlockSpec((2, BM, 128), lambda i: (0, i, 0)),
            pl.BlockSpec((2, BM, 128), lambda i: (0, i, 0)),
            pl.BlockSpec((BM, D_HID), lambda i: (i, 0)),
            pl.BlockSpec((D_HID, 2 * D_OUT), lambda i: (0, 0)),
            pl.BlockSpec((1, D_OUT), lambda i: (0, 0)),
        ],
        out_specs=[
            pl.BlockSpec((BM, D_OUT), lambda i: (i, 0)),
            pl.BlockSpec((BM, D_OUT), lambda i: (i, 0)),
        ],
        out_shape=[
            jax.ShapeDtypeStruct((N, D_OUT), jnp.float32),
            jax.ShapeDtypeStruct((N, D_OUT), jnp.float32),
        ],
    )(s1a, s1b, degp, v1, wcat, b2)


def _tc3(s2, degp, v2):
    def body(s2_ref, deg_ref, v2_ref, out_ref):
        deg = jnp.maximum(deg_ref[0, :, :1] + deg_ref[1, :, :1], 1.0)
        z = (s2_ref[0] + s2_ref[1]) / deg + v2_ref[...]
        m = jnp.max(z, axis=1, keepdims=True)
        lse = jnp.log(jnp.sum(jnp.exp(z - m), axis=1, keepdims=True))
        out_ref[...] = z - m - lse

    return pl.pallas_call(
        body,
        grid=(N // BM,),
        in_specs=[
            pl.BlockSpec((2, BM, D_OUT), lambda i: (0, i, 0)),
            pl.BlockSpec((2, BM, 128), lambda i: (0, i, 0)),
            pl.BlockSpec((BM, D_OUT), lambda i: (i, 0)),
        ],
        out_specs=pl.BlockSpec((BM, D_OUT), lambda i: (i, 0)),
        out_shape=jax.ShapeDtypeStruct((N, D_OUT), jnp.float32),
    )(s2, degp, v2)


def kernel(x, edge_index, W1_l, W1_r, b1, W2_l, W2_r, b2):
    src = edge_index[0].astype(jnp.int32)
    dst = edge_index[1].astype(jnp.int32)

    he = E // NC
    pad = NS * NB2 * BATCH - he
    zp = jnp.zeros((pad,), jnp.int32)
    fp = jnp.full((pad,), N, jnp.int32)
    srcp = jnp.concatenate([src[:he], zp, src[he:], zp]).reshape(
        NC, NS, NB2, BATCH)
    dstp = jnp.concatenate([dst[:he], fp, dst[he:], fp]).reshape(
        NC, NS, NB2, BATCH)

    zeros = jnp.zeros((NPAD, 128), jnp.float32)
    ones = jnp.ones((BATCH, 128), jnp.float32)

    segsum = _make_segsum(NB2)
    u1, v1 = _tc1(x, jnp.concatenate([W1_l, W1_r], axis=1),
                  b1.reshape(1, -1))
    (degp,) = _make_deg()(dstp, zeros, ones)
    (s1a,) = segsum(u1[0], srcp, dstp, zeros)
    (s1b,) = segsum(u1[1], srcp, dstp, zeros)
    u2, v2 = _tc2(s1a, s1b, degp, v1,
                  jnp.concatenate([W2_l, W2_r], axis=1), b2.reshape(1, -1))
    (s2,) = segsum(u2, srcp, dstp, zeros)
    return _tc3(s2, degp, v2)


# speedup vs baseline: 1.1424x; 1.1424x over previous
import functools

import jax
import jax.numpy as jnp
from jax import lax
from jax.experimental import pallas as pl
from jax.experimental.pallas import tpu as pltpu
from jax.experimental.pallas import tpu_sc as plsc

N = 10000
E = 160000
D_IN = 256
D_HID = 256
D_OUT = 128
NC, NS = 2, 16
RPS = 632
NPAD = RPS * NS
BATCH = 128
PB = 40
NB1 = 80
NB2 = 40
BM = 1000


@functools.lru_cache(maxsize=None)
def _make_segsum(nb):
    mesh = plsc.VectorSubcoreMesh(core_axis_name="c", subcore_axis_name="s",
                                  num_cores=NC, num_subcores=NS)

    def body(table, srcp, dstp, zeros, out, src_v, dst_v, rows_a, rows_b,
             acc, sem_a, sem_b):
        c = lax.axis_index("c")
        s = lax.axis_index("s")
        sl = pl.ds(s * RPS, RPS)

        def phase(t, first):
            pltpu.sync_copy(srcp.at[c, s, pl.ds(t * PB, PB)], src_v)
            pltpu.sync_copy(dstp.at[c, s, pl.ds(t * PB, PB)], dst_v)
            pltpu.async_copy(table.at[src_v.at[0]], rows_a, sem_a)
            pltpu.async_copy(table.at[src_v.at[1]], rows_b, sem_b)
            if first:
                pltpu.sync_copy(zeros.at[sl], acc.at[sl])
                plsc.subcore_barrier()

            def step(i, carry):
                ja = 2 * i
                jb = ja + 1
                pltpu.make_async_copy(table.at[src_v.at[ja]], rows_a,
                                      sem_a).wait()
                pltpu.sync_copy(rows_a, acc.at[dst_v.at[ja]], add=True)

                @pl.when(ja + 2 < PB)
                def _():
                    pltpu.async_copy(table.at[src_v.at[ja + 2]], rows_a,
                                     sem_a)

                pltpu.make_async_copy(table.at[src_v.at[jb]], rows_b,
                                      sem_b).wait()
                pltpu.sync_copy(rows_b, acc.at[dst_v.at[jb]], add=True)

                @pl.when(jb + 2 < PB)
                def _():
                    pltpu.async_copy(table.at[src_v.at[jb + 2]], rows_b,
                                     sem_b)

                return carry

            lax.fori_loop(0, PB // 2, step, 0)

        for t in range(nb // PB):
            phase(t, t == 0)
        plsc.subcore_barrier()
        pltpu.sync_copy(acc.at[sl], out.at[c, sl])

    return pl.kernel(
        body,
        out_type=[jax.ShapeDtypeStruct((NC, NPAD, 128), jnp.float32)],
        mesh=mesh,
        scratch_types=[
            pltpu.VMEM((PB, BATCH), jnp.int32),
            pltpu.VMEM((PB, BATCH), jnp.int32),
            pltpu.VMEM((BATCH, 128), jnp.float32),
            pltpu.VMEM((BATCH, 128), jnp.float32),
            pltpu.VMEM_SHARED((NPAD, 128), jnp.float32),
            pltpu.SemaphoreType.DMA,
            pltpu.SemaphoreType.DMA,
        ])


@functools.lru_cache(maxsize=None)
def _make_deg():
    mesh = plsc.VectorSubcoreMesh(core_axis_name="c", subcore_axis_name="s",
                                  num_cores=NC, num_subcores=NS)

    def body(dstp, zeros, ones, degp, dst_v, ones_v, dacc):
        c = lax.axis_index("c")
        s = lax.axis_index("s")
        sl = pl.ds(s * RPS, RPS)
        pltpu.sync_copy(dstp.at[c, s], dst_v)
        pltpu.sync_copy(ones, ones_v)
        pltpu.sync_copy(zeros.at[sl], dacc.at[sl])
        plsc.subcore_barrier()

        def step(j, carry):
            pltpu.sync_copy(ones_v, dacc.at[dst_v.at[j]], add=True)
            return carry

        lax.fori_loop(0, NB2, step, 0)
        plsc.subcore_barrier()
        pltpu.sync_copy(dacc.at[sl], degp.at[c, sl])

    return pl.kernel(
        body,
        out_type=[jax.ShapeDtypeStruct((NC, NPAD, 128), jnp.float32)],
        mesh=mesh,
        scratch_types=[
            pltpu.VMEM((NB2, BATCH), jnp.int32),
            pltpu.VMEM((BATCH, 128), jnp.float32),
            pltpu.VMEM_SHARED((NPAD, 128), jnp.float32),
        ])


def _tc1(x, wcat, b1):
    def body(x_ref, w_ref, b_ref, u_ref, v_ref):
        y = jnp.dot(x_ref[...], w_ref[...],
                    preferred_element_type=jnp.float32)
        u_ref[0] = y[:, :128]
        u_ref[1] = y[:, 128:256]
        v_ref[...] = y[:, 256:] + b_ref[...]

    return pl.pallas_call(
        body,
        grid=(N // BM,),
        in_specs=[
            pl.BlockSpec((BM, D_IN), lambda i: (i, 0)),
            pl.BlockSpec((D_IN, 2 * D_HID), lambda i: (0, 0)),
            pl.BlockSpec((1, D_HID), lambda i: (0, 0)),
        ],
        out_specs=[
            pl.BlockSpec((2, BM, 128), lambda i: (0, i, 0)),
            pl.BlockSpec((BM, D_HID), lambda i: (i, 0)),
        ],
        out_shape=[
            jax.ShapeDtypeStruct((2, N, 128), jnp.float32),
            jax.ShapeDtypeStruct((N, D_HID), jnp.float32),
        ],
    )(x, wcat, b1)


def _tc2(s1, degp, v1, wcat, b2):
    def body(s1_ref, deg_ref, v1_ref, w_ref, b_ref, u2_ref, v2_ref):
        deg = jnp.maximum(deg_ref[0, :, :1] + deg_ref[1, :, :1], 1.0)
        agg = jnp.concatenate([s1_ref[0], s1_ref[1]], axis=1) / deg
        h = jnp.maximum(agg + v1_ref[...], 0.0)
        y = jnp.dot(h, w_ref[...], preferred_element_type=jnp.float32)
        u2_ref[...] = y[:, :D_OUT]
        v2_ref[...] = y[:, D_OUT:] + b_ref[...]

    return pl.pallas_call(
        body,
        grid=(N // BM,),
        in_specs=[
            pl.BlockSpec((2, BM, 128), lambda i: (0, i, 0)),
            pl.BlockSpec((2, BM, 128), lambda i: (0, i, 0)),
            pl.BlockSpec((BM, D_HID), lambda i: (i, 0)),
            pl.BlockSpec((D_HID, 2 * D_OUT), lambda i: (0, 0)),
            pl.BlockSpec((1, D_OUT), lambda i: (0, 0)),
        ],
        out_specs=[
            pl.BlockSpec((BM, D_OUT), lambda i: (i, 0)),
            pl.BlockSpec((BM, D_OUT), lambda i: (i, 0)),
        ],
        out_shape=[
            jax.ShapeDtypeStruct((N, D_OUT), jnp.float32),
            jax.ShapeDtypeStruct((N, D_OUT), jnp.float32),
        ],
    )(s1, degp, v1, wcat, b2)


def _tc3(s2, degp, v2):
    def body(s2_ref, deg_ref, v2_ref, out_ref):
        deg = jnp.maximum(deg_ref[0, :, :1] + deg_ref[1, :, :1], 1.0)
        z = (s2_ref[0] + s2_ref[1]) / deg + v2_ref[...]
        m = jnp.max(z, axis=1, keepdims=True)
        lse = jnp.log(jnp.sum(jnp.exp(z - m), axis=1, keepdims=True))
        out_ref[...] = z - m - lse

    return pl.pallas_call(
        body,
        grid=(N // BM,),
        in_specs=[
            pl.BlockSpec((2, BM, D_OUT), lambda i: (0, i, 0)),
            pl.BlockSpec((2, BM, 128), lambda i: (0, i, 0)),
            pl.BlockSpec((BM, D_OUT), lambda i: (i, 0)),
        ],
        out_specs=pl.BlockSpec((BM, D_OUT), lambda i: (i, 0)),
        out_shape=jax.ShapeDtypeStruct((N, D_OUT), jnp.float32),
    )(s2, degp, v2)


def kernel(x, edge_index, W1_l, W1_r, b1, W2_l, W2_r, b2):
    src = edge_index[0].astype(jnp.int32)
    dst = edge_index[1].astype(jnp.int32)

    pad1 = NS * NB1 * BATCH - E
    s1p = jnp.concatenate([src, jnp.zeros((pad1,), jnp.int32)])
    d1p = jnp.concatenate([dst, jnp.full((pad1,), N, jnp.int32)])
    srcp1 = jnp.stack([s1p, s1p + N]).reshape(NC, NS, NB1, BATCH)
    dstp1 = jnp.stack([d1p, d1p]).reshape(NC, NS, NB1, BATCH)

    he = E // NC
    pad2 = NS * NB2 * BATCH - he
    zp = jnp.zeros((pad2,), jnp.int32)
    fp = jnp.full((pad2,), N, jnp.int32)
    srcp2 = jnp.concatenate([src[:he], zp, src[he:], zp]).reshape(
        NC, NS, NB2, BATCH)
    dstp2 = jnp.concatenate([dst[:he], fp, dst[he:], fp]).reshape(
        NC, NS, NB2, BATCH)

    zeros = jnp.zeros((NPAD, 128), jnp.float32)
    ones = jnp.ones((BATCH, 128), jnp.float32)

    u1, v1 = _tc1(x, jnp.concatenate([W1_l, W1_r], axis=1),
                  b1.reshape(1, -1))
    (degp,) = _make_deg()(dstp2, zeros, ones)
    (s1,) = _make_segsum(NB1)(u1.reshape(2 * N, 128), srcp1, dstp1, zeros)
    u2, v2 = _tc2(s1, degp, v1,
                  jnp.concatenate([W2_l, W2_r], axis=1), b2.reshape(1, -1))
    (s2,) = _make_segsum(NB2)(u2, srcp2, dstp2, zeros)
    return _tc3(s2, degp, v2)


# speedup vs baseline: 1.1434x; 1.0009x over previous
import functools

import jax
import jax.numpy as jnp
from jax import lax
from jax.experimental import pallas as pl
from jax.experimental.pallas import tpu as pltpu
from jax.experimental.pallas import tpu_sc as plsc

N = 10000
E = 160000
D_IN = 256
D_HID = 256
D_OUT = 128
NC, NS = 2, 16
RPS = 632
NPAD = RPS * NS
BATCH = 128
PB = 40
NB1 = 80
NB2 = 40
BM = 1000


@functools.lru_cache(maxsize=None)
def _make_segsum(nb):
    mesh = plsc.VectorSubcoreMesh(core_axis_name="c", subcore_axis_name="s",
                                  num_cores=NC, num_subcores=NS)

    def body(table, srcp, dstp, zeros, out, src_v, dst_v, rows_a, rows_b,
             acc, sem_a, sem_b):
        c = lax.axis_index("c")
        s = lax.axis_index("s")
        sl = pl.ds(s * RPS, RPS)

        def phase(t, first):
            pltpu.sync_copy(srcp.at[c, s, pl.ds(t * PB, PB)], src_v)
            pltpu.sync_copy(dstp.at[c, s, pl.ds(t * PB, PB)], dst_v)
            pltpu.async_copy(table.at[src_v.at[0]], rows_a, sem_a)
            pltpu.async_copy(table.at[src_v.at[1]], rows_b, sem_b)
            if first:
                pltpu.sync_copy(zeros.at[sl], acc.at[sl])
                plsc.subcore_barrier()

            def step(i, carry):
                ja = 2 * i
                jb = ja + 1
                pltpu.make_async_copy(table.at[src_v.at[ja]], rows_a,
                                      sem_a).wait()
                pltpu.sync_copy(rows_a, acc.at[dst_v.at[ja]], add=True)

                @pl.when(ja + 2 < PB)
                def _():
                    pltpu.async_copy(table.at[src_v.at[ja + 2]], rows_a,
                                     sem_a)

                pltpu.make_async_copy(table.at[src_v.at[jb]], rows_b,
                                      sem_b).wait()
                pltpu.sync_copy(rows_b, acc.at[dst_v.at[jb]], add=True)

                @pl.when(jb + 2 < PB)
                def _():
                    pltpu.async_copy(table.at[src_v.at[jb + 2]], rows_b,
                                     sem_b)

                return carry

            lax.fori_loop(0, PB // 2, step, 0)

        for t in range(nb // PB):
            phase(t, t == 0)
        plsc.subcore_barrier()
        pltpu.sync_copy(acc.at[sl], out.at[c, sl])

    return pl.kernel(
        body,
        out_type=[jax.ShapeDtypeStruct((NC, NPAD, 128), jnp.float32)],
        mesh=mesh,
        scratch_types=[
            pltpu.VMEM((PB, BATCH), jnp.int32),
            pltpu.VMEM((PB, BATCH), jnp.int32),
            pltpu.VMEM((BATCH, 128), jnp.float32),
            pltpu.VMEM((BATCH, 128), jnp.float32),
            pltpu.VMEM_SHARED((NPAD, 128), jnp.float32),
            pltpu.SemaphoreType.DMA,
            pltpu.SemaphoreType.DMA,
        ])


@functools.lru_cache(maxsize=None)
def _make_deg():
    mesh = plsc.VectorSubcoreMesh(core_axis_name="c", subcore_axis_name="s",
                                  num_cores=NC, num_subcores=NS)

    def body(dstp, zeros, ones, degp, dst_v, ones_v, dacc):
        c = lax.axis_index("c")
        s = lax.axis_index("s")
        sl = pl.ds(s * RPS, RPS)
        pltpu.sync_copy(dstp.at[c, s], dst_v)
        pltpu.sync_copy(ones, ones_v)
        pltpu.sync_copy(zeros.at[sl], dacc.at[sl])
        plsc.subcore_barrier()

        def step(j, carry):
            pltpu.sync_copy(ones_v, dacc.at[dst_v.at[j]], add=True)
            return carry

        lax.fori_loop(0, NB2, step, 0)
        plsc.subcore_barrier()
        pltpu.sync_copy(dacc.at[sl], degp.at[c, sl])

    return pl.kernel(
        body,
        out_type=[jax.ShapeDtypeStruct((NC, NPAD, 128), jnp.float32)],
        mesh=mesh,
        scratch_types=[
            pltpu.VMEM((NB2, BATCH), jnp.int32),
            pltpu.VMEM((BATCH, 128), jnp.float32),
            pltpu.VMEM_SHARED((NPAD, 128), jnp.float32),
        ])


def _tc1(x, wcat, b1):
    def body(x_ref, w_ref, b_ref, u_ref, v_ref):
        y = jnp.dot(x_ref[...], w_ref[...],
                    preferred_element_type=jnp.float32)
        u_ref[0] = y[:, :128]
        u_ref[1] = y[:, 128:256]
        v_ref[...] = y[:, 256:] + b_ref[...]

    return pl.pallas_call(
        body,
        grid=(N // BM,),
        in_specs=[
            pl.BlockSpec((BM, D_IN), lambda i: (i, 0)),
            pl.BlockSpec((D_IN, 2 * D_HID), lambda i: (0, 0)),
            pl.BlockSpec((1, D_HID), lambda i: (0, 0)),
        ],
        out_specs=[
            pl.BlockSpec((2, BM, 128), lambda i: (0, i, 0)),
            pl.BlockSpec((BM, D_HID), lambda i: (i, 0)),
        ],
        out_shape=[
            jax.ShapeDtypeStruct((2, N, 128), jnp.float32),
            jax.ShapeDtypeStruct((N, D_HID), jnp.float32),
        ],
    )(x, wcat, b1)


def _tc2(s1, degp, v1, wcat, b2):
    def body(s1_ref, deg_ref, v1_ref, w_ref, b_ref, u2_ref, v2_ref):
        deg = jnp.maximum(deg_ref[0, :, :1] + deg_ref[1, :, :1], 1.0)
        agg = jnp.concatenate([s1_ref[0], s1_ref[1]], axis=1) / deg
        h = jnp.maximum(agg + v1_ref[...], 0.0)
        y = jnp.dot(h, w_ref[...], preferred_element_type=jnp.float32)
        u2_ref[...] = y[:, :D_OUT]
        v2_ref[...] = y[:, D_OUT:] + b_ref[...]

    return pl.pallas_call(
        body,
        grid=(N // BM,),
        in_specs=[
            pl.BlockSpec((2, BM, 128), lambda i: (0, i, 0)),
            pl.BlockSpec((2, BM, 128), lambda i: (0, i, 0)),
            pl.BlockSpec((BM, D_HID), lambda i: (i, 0)),
            pl.BlockSpec((D_HID, 2 * D_OUT), lambda i: (0, 0)),
            pl.BlockSpec((1, D_OUT), lambda i: (0, 0)),
        ],
        out_specs=[
            pl.BlockSpec((BM, D_OUT), lambda i: (i, 0)),
            pl.BlockSpec((BM, D_OUT), lambda i: (i, 0)),
        ],
        out_shape=[
            jax.ShapeDtypeStruct((N, D_OUT), jnp.float32),
            jax.ShapeDtypeStruct((N, D_OUT), jnp.float32),
        ],
    )(s1, degp, v1, wcat, b2)


def _tc3(s2, degp, v2):
    def body(s2_ref, deg_ref, v2_ref, out_ref):
        deg = jnp.maximum(deg_ref[0, :, :1] + deg_ref[1, :, :1], 1.0)
        z = (s2_ref[0] + s2_ref[1]) / deg + v2_ref[...]
        m = jnp.max(z, axis=1, keepdims=True)
        lse = jnp.log(jnp.sum(jnp.exp(z - m), axis=1, keepdims=True))
        out_ref[...] = z - m - lse

    return pl.pallas_call(
        body,
        grid=(N // BM,),
        in_specs=[
            pl.BlockSpec((2, BM, D_OUT), lambda i: (0, i, 0)),
            pl.BlockSpec((2, BM, 128), lambda i: (0, i, 0)),
            pl.BlockSpec((BM, D_OUT), lambda i: (i, 0)),
        ],
        out_specs=pl.BlockSpec((BM, D_OUT), lambda i: (i, 0)),
        out_shape=jax.ShapeDtypeStruct((N, D_OUT), jnp.float32),
    )(s2, degp, v2)


def kernel(x, edge_index, W1_l, W1_r, b1, W2_l, W2_r, b2):
    src = edge_index[0].astype(jnp.int32)
    dst = edge_index[1].astype(jnp.int32)

    pad1 = NS * NB1 * BATCH - E
    s1p = jnp.concatenate([src, jnp.zeros((pad1,), jnp.int32)])
    d1p = jnp.concatenate([dst, jnp.full((pad1,), N, jnp.int32)])
    srcp1 = jnp.stack([s1p, s1p + N]).reshape(NC, NS, NB1, BATCH)
    dstp1 = jnp.stack([d1p, d1p]).reshape(NC, NS, NB1, BATCH)

    he = E // NC
    pad2 = NS * NB2 * BATCH - he
    zp = jnp.zeros((pad2,), jnp.int32)
    fp = jnp.full((pad2,), N, jnp.int32)
    srcp2 = jnp.concatenate([src[:he], zp, src[he:], zp]).reshape(
        NC, NS, NB2, BATCH)
    dstp2 = jnp.concatenate([dst[:he], fp, dst[he:], fp]).reshape(
        NC, NS, NB2, BATCH)

    zeros = jnp.zeros((NPAD, 128), jnp.float32)
    ones = jnp.ones((BATCH, 128), jnp.float32)

    (degp,) = _make_deg()(dstp2, zeros, ones)
    u1, v1 = _tc1(x, jnp.concatenate([W1_l, W1_r], axis=1),
                  b1.reshape(1, -1))
    (s1,) = _make_segsum(NB1)(u1.reshape(2 * N, 128), srcp1, dstp1, zeros)
    u2, v2 = _tc2(s1, degp, v1,
                  jnp.concatenate([W2_l, W2_r], axis=1), b2.reshape(1, -1))
    (s2,) = _make_segsum(NB2)(u2, srcp2, dstp2, zeros)
    return _tc3(s2, degp, v2)
